# Initial kernel scaffold; baseline (speedup 1.0000x reference)
import jax
import jax.numpy as jnp
from jax.experimental import pallas as pl


def kernel(x, experts, gate_w, gate_b):
    raise NotImplementedError("write your pallas kernel here")


# speedup vs baseline: 3.1534x; 3.1534x over previous
import functools

import jax
import jax.numpy as jnp
from jax.experimental import pallas as pl
from jax.experimental.pallas import tpu as pltpu


def _moe_kernel(x_ref, ex_ref, gw_ref, gb_ref, out_ref, wt_ref, *, k, sb):
    s_idx = pl.program_id(1)

    @pl.when(s_idx == 0)
    def _gate():
        xb = x_ref[0]
        gw = gw_ref[...]
        gb = gb_ref[...]
        e = gw.shape[0]
        hw = xb.shape[1]
        logits = jnp.dot(gw, xb, preferred_element_type=jnp.float32) + gb
        m = jnp.max(logits, axis=0, keepdims=True)
        p = jnp.exp(logits - m)
        rw = p / jnp.sum(p, axis=0, keepdims=True)

        rows = jax.lax.broadcasted_iota(jnp.int32, (e, hw), 0)
        cur = rw
        wsel = jnp.zeros_like(rw)
        for _ in range(k):
            mx = jnp.max(cur, axis=0, keepdims=True)
            sel = cur == mx
            first = jnp.min(jnp.where(sel, rows, e), axis=0, keepdims=True)
            sel = rows == first
            wsel = wsel + jnp.where(sel, rw, 0.0)
            cur = jnp.where(sel, -1.0, cur)
        wt_ref[...] = wsel.T

    e = gw_ref.shape[0]
    wt = wt_ref[pl.ds(s_idx * sb, sb), :]
    acc = wt[:, 0:1] * ex_ref[0, 0]
    for j in range(1, e):
        acc = acc + wt[:, j:j + 1] * ex_ref[0, j]
    out_ref[0] = acc


def kernel(x, experts, gate_w, gate_b):
    n, c, h, w = x.shape
    _, e, _, _, d = experts.shape
    k = 4
    hw = h * w
    sb = 448
    nsb = hw // sb

    xr = x.reshape(n, c, hw)
    er = experts.reshape(n, e, hw, d)
    gb = gate_b.reshape(e, 1)

    out = pl.pallas_call(
        functools.partial(_moe_kernel, k=k, sb=sb),
        grid=(n, nsb),
        in_specs=[
            pl.BlockSpec((1, c, hw), lambda i, s: (i, 0, 0)),
            pl.BlockSpec((1, e, sb, d), lambda i, s: (i, 0, s, 0)),
            pl.BlockSpec((e, c), lambda i, s: (0, 0)),
            pl.BlockSpec((e, 1), lambda i, s: (0, 0)),
        ],
        out_specs=pl.BlockSpec((1, sb, d), lambda i, s: (i, s, 0)),
        out_shape=jax.ShapeDtypeStruct((n, hw, d), jnp.float32),
        scratch_shapes=[pltpu.VMEM((hw, e), jnp.float32)],
    )(xr, er, gate_w, gb)
    return out.reshape(n, h, w, d)
